# baseline (speedup 1.0000x reference)
import functools

import jax
import jax.numpy as jnp
from jax import lax
from jax.experimental import pallas as pl
from jax.experimental.pallas import tpu as pltpu
from jax.experimental.pallas import tpu_sc as plsc

_N = 50000
_E = 800000
_NH = 32
_DE = 16

_EH0 = 403200
_EH1 = _E - _EH0

_NC = 2
_NS = 16
_NW = _NC * _NS

_SCH = 128

_NPS = _N // _NS

def _sc_gather(s_tab, v_tab, i_idx, j_idx):
    eh = i_idx.shape[0]
    epw = eh // _NW
    sfull = epw // _SCH
    stail = epw - sfull * _SCH

    def body(s_tab, v_tab, i_hbm, j_hbm, out_si, out_sj, out_vi, out_vj,
             *scr):
        idxi, idxj, si_v, sj_v, vi_v, vj_v = scr[0:6]
        sem1, sem2, sem3, sem4 = scr[-4:]
        c = lax.axis_index("c")
        sc = lax.axis_index("s")
        wid = sc * _NC + c
        e0 = wid * epw

        def step(base, idxi, idxj, si_v, sj_v, vi_v, vj_v, n):
            pltpu.sync_copy(i_hbm.at[pl.ds(base, n)], idxi)
            pltpu.sync_copy(j_hbm.at[pl.ds(base, n)], idxj)
            cp1 = pltpu.async_copy(s_tab.at[idxi], si_v, sem1)
            cp2 = pltpu.async_copy(s_tab.at[idxj], sj_v, sem2)
            cp3 = pltpu.async_copy(v_tab.at[idxi], vi_v, sem3)
            cp4 = pltpu.async_copy(v_tab.at[idxj], vj_v, sem4)
            cp1.wait()
            cp2.wait()
            cp3.wait()
            cp4.wait()
            pltpu.sync_copy(si_v, out_si.at[pl.ds(base, n)])
            pltpu.sync_copy(sj_v, out_sj.at[pl.ds(base, n)])
            pltpu.sync_copy(vi_v, out_vi.at[pl.ds(base, n)])
            pltpu.sync_copy(vj_v, out_vj.at[pl.ds(base, n)])

        def chunk(k, carry):
            step(e0 + _SCH * k, idxi, idxj, si_v, sj_v, vi_v, vj_v, _SCH)
            return carry

        lax.fori_loop(0, sfull, chunk, 0)
        if stail:
            step(e0 + _SCH * sfull, *scr[6:12], stail)

    scratch = [
        pltpu.VMEM((_SCH,), jnp.int32),
        pltpu.VMEM((_SCH,), jnp.int32),
        pltpu.VMEM((_SCH, _NH), jnp.float32),
        pltpu.VMEM((_SCH, _NH), jnp.float32),
        pltpu.VMEM((_SCH, 16), jnp.float32),
        pltpu.VMEM((_SCH, 16), jnp.float32),
    ]
    if stail:
        scratch += [
            pltpu.VMEM((stail,), jnp.int32),
            pltpu.VMEM((stail,), jnp.int32),
            pltpu.VMEM((stail, _NH), jnp.float32),
            pltpu.VMEM((stail, _NH), jnp.float32),
            pltpu.VMEM((stail, 16), jnp.float32),
            pltpu.VMEM((stail, 16), jnp.float32),
        ]
    scratch += [pltpu.SemaphoreType.DMA] * 4

    mesh = plsc.VectorSubcoreMesh(core_axis_name="c", subcore_axis_name="s")
    f = pl.kernel(
        body,
        out_type=(
            jax.ShapeDtypeStruct((eh, _NH), jnp.float32),
            jax.ShapeDtypeStruct((eh, _NH), jnp.float32),
            jax.ShapeDtypeStruct((eh, 16), jnp.float32),
            jax.ShapeDtypeStruct((eh, 16), jnp.float32),
        ),
        mesh=mesh,
        scratch_types=scratch,
        compiler_params=pltpu.CompilerParams(use_tc_tiling_on_sc=False),
    )
    return f(s_tab, v_tab, i_idx, j_idx)


def _sc_scatter(mu, i_idx, zeros, width):
    eh = i_idx.shape[0]
    epw = eh // _NW
    sfull = epw // _SCH
    stail = epw - sfull * _SCH

    def body(mu_hbm, i_hbm, z_hbm, out_hbm, *scr):
        idx_v, rows_v = scr[0:2]
        acc = scr[-1]
        c = lax.axis_index("c")
        sc = lax.axis_index("s")
        wid = sc * _NC + c
        e0 = wid * epw

        pltpu.sync_copy(z_hbm.at[pl.ds(sc * _NPS, _NPS)],
                        acc.at[pl.ds(sc * _NPS, _NPS)])
        plsc.subcore_barrier()

        def step(base, idx, rows, n):
            pltpu.sync_copy(i_hbm.at[pl.ds(base, n)], idx)
            pltpu.sync_copy(mu_hbm.at[pl.ds(base, n)], rows)
            pltpu.sync_copy(rows, acc.at[idx], add=True)

        def chunk(k, carry):
            step(e0 + _SCH * k, idx_v, rows_v, _SCH)
            return carry

        lax.fori_loop(0, sfull, chunk, 0)
        if stail:
            step(e0 + _SCH * sfull, scr[2], scr[3], stail)

        plsc.subcore_barrier()
        pltpu.sync_copy(acc.at[pl.ds(sc * _NPS, _NPS)],
                        out_hbm.at[c].at[pl.ds(sc * _NPS, _NPS)])

    scratch = [
        pltpu.VMEM((_SCH,), jnp.int32),
        pltpu.VMEM((_SCH, width), jnp.float32),
    ]
    if stail:
        scratch += [
            pltpu.VMEM((stail,), jnp.int32),
            pltpu.VMEM((stail, width), jnp.float32),
        ]
    scratch += [pltpu.VMEM_SHARED((_N, width), jnp.float32)]

    mesh = plsc.VectorSubcoreMesh(core_axis_name="c", subcore_axis_name="s")
    f = pl.kernel(
        body,
        out_type=jax.ShapeDtypeStruct((_NC, _N, width), jnp.float32),
        mesh=mesh,
        scratch_types=scratch,
        compiler_params=pltpu.CompilerParams(use_tc_tiling_on_sc=False),
    )
    return f(mu, i_idx, zeros)


def _lnorm(x, g, b):
    mu = jnp.mean(x, axis=-1, keepdims=True)
    var = jnp.mean(x * x, axis=-1, keepdims=True) - mu * mu
    return (x - mu) / jnp.sqrt(var + 1e-5) * g + b


def _lnorm_fm(x, g, b):
    f = x.shape[0]
    ones = jnp.full((1, f), 1.0 / f, jnp.float32)
    dot = functools.partial(jnp.dot, preferred_element_type=jnp.float32)
    mu = dot(ones, x)
    var = dot(ones, x * x) - mu * mu
    return (x - mu) * lax.rsqrt(var + 1e-5) * g + b


def _dense_body(si_r, sj_r, vi_r, vj_r, eT_r,
                w0siT, w0sjT, w0eT, w0ndT, b0, g0, be0, w1T, b1,
                mw0T, mb0, mg0, mbe0, mw1T, mb1,
                xw0T, xb0, xg0, xbe0, xw1T, xb1, xg1, xbe1, xw2T, xb2,
                eijT_o, m_o, u_o):
    f32 = jnp.float32
    dot = functools.partial(jnp.dot, preferred_element_type=f32)
    att = lambda w, x: lax.dot_general(w[...], x[...], (((1,), (1,)), ((), ())),
                                       preferred_element_type=f32)

    viT = vi_r[...].T
    vjT = vj_r[...].T
    vi3 = viT[0:3]
    vj3 = vjT[0:3]
    vd = vi3 - vj3
    nsq = jnp.sum(vd * vd, axis=0, keepdims=True) + 1e-8
    norms = jnp.sqrt(nsq)
    dots = jnp.sum(vi3 * vj3, axis=0, keepdims=True)
    nd = jnp.concatenate([norms, dots], axis=0)

    pre = (att(w0siT, si_r) + att(w0sjT, sj_r)
           + dot(w0eT[...], eT_r[...]) + dot(w0ndT[...], nd) + b0[...])
    h = jnp.maximum(_lnorm_fm(pre, g0[...], be0[...]), 0.0)
    eij = dot(w1T[...], h) + b1[...]
    eijT_o[...] = eij

    h = jnp.maximum(_lnorm_fm(dot(mw0T[...], eij) + mb0[...], mg0[...], mbe0[...]), 0.0)
    m = jax.nn.sigmoid(dot(mw1T[...], h) + mb1[...])
    m_o[...] = m.T

    h = jnp.maximum(_lnorm_fm(dot(xw0T[...], m) + xb0[...], xg0[...], xbe0[...]), 0.0)
    h = jnp.maximum(_lnorm_fm(dot(xw1T[...], h) + xb1[...], xg1[...], xbe1[...]), 0.0)
    w = dot(xw2T[...], h) + xb2[...]
    upd = jnp.clip(vd * w, -100.0, 100.0)
    u8 = jnp.concatenate([upd, jnp.zeros((5, upd.shape[1]), f32)], axis=0)
    u_o[...] = u8.T


_TILE_E = 3200


def _tc_dense(si, sj, vi, vj, eT, wlist):
    ne = si.shape[0]
    grid = ne // _TILE_E
    full = lambda a: pl.BlockSpec(a.shape, lambda i: (0, 0))
    in_specs = [
        pl.BlockSpec((_TILE_E, _NH), lambda i: (i, 0)),
        pl.BlockSpec((_TILE_E, _NH), lambda i: (i, 0)),
        pl.BlockSpec((_TILE_E, 16), lambda i: (i, 0)),
        pl.BlockSpec((_TILE_E, 16), lambda i: (i, 0)),
        pl.BlockSpec((_DE, _TILE_E), lambda i: (0, i)),
    ] + [full(w) for w in wlist]
    out_specs = [
        pl.BlockSpec((_NH, _TILE_E), lambda i: (0, i)),
        pl.BlockSpec((_TILE_E, _NH), lambda i: (i, 0)),
        pl.BlockSpec((_TILE_E, 8), lambda i: (i, 0)),
    ]
    return pl.pallas_call(
        _dense_body,
        grid=(grid,),
        in_specs=in_specs,
        out_specs=out_specs,
        out_shape=[
            jax.ShapeDtypeStruct((_NH, ne), jnp.float32),
            jax.ShapeDtypeStruct((ne, _NH), jnp.float32),
            jax.ShapeDtypeStruct((ne, 8), jnp.float32),
        ],
        compiler_params=pltpu.CompilerParams(
            dimension_semantics=("arbitrary",),
        ),
    )(si, sj, vi, vj, eT, *wlist)


def _node_body(v_r, s_r, pm0_r, pm1_r, pu0_r, pu1_r,
               sw0a, sw0b, sb0, sg0, sbe0, sw1, sb1,
               vt_o, st_o):
    f32 = jnp.float32
    s_agg = pm0_r[0] + pm0_r[1] + pm1_r[0] + pm1_r[1]
    x_agg = (pu0_r[0] + pu0_r[1] + pu1_r[0] + pu1_r[1])[:, 0:3]
    s = s_r[...]
    dotf = functools.partial(jnp.dot, preferred_element_type=f32)
    pre = dotf(s, sw0a[...]) + dotf(s_agg, sw0b[...]) + sb0[...]
    h = jnp.maximum(_lnorm(pre, sg0[...], sbe0[...]), 0.0)
    st_o[...] = s + dotf(h, sw1[...]) + sb1[...]
    vt_o[...] = v_r[...] + x_agg


_TILE_N = 2000


def _tc_node(v, s, pm0, pm1, pu0, pu1, wlist):
    grid = _N // _TILE_N
    full = lambda a: pl.BlockSpec(a.shape, lambda i: (0, 0))
    in_specs = [
        pl.BlockSpec((_TILE_N, 3), lambda i: (i, 0)),
        pl.BlockSpec((_TILE_N, _NH), lambda i: (i, 0)),
        pl.BlockSpec((_NC, _TILE_N, _NH), lambda i: (0, i, 0)),
        pl.BlockSpec((_NC, _TILE_N, _NH), lambda i: (0, i, 0)),
        pl.BlockSpec((_NC, _TILE_N, 8), lambda i: (0, i, 0)),
        pl.BlockSpec((_NC, _TILE_N, 8), lambda i: (0, i, 0)),
    ] + [full(w) for w in wlist]
    out_specs = [
        pl.BlockSpec((_TILE_N, 3), lambda i: (i, 0)),
        pl.BlockSpec((_TILE_N, _NH), lambda i: (i, 0)),
    ]
    return pl.pallas_call(
        _node_body,
        grid=(grid,),
        in_specs=in_specs,
        out_specs=out_specs,
        out_shape=[
            jax.ShapeDtypeStruct((_N, 3), jnp.float32),
            jax.ShapeDtypeStruct((_N, _NH), jnp.float32),
        ],
        compiler_params=pltpu.CompilerParams(
            dimension_semantics=("arbitrary",),
        ),
    )(v, s, pm0, pm1, pu0, pu1, *wlist)


def kernel(v, edge_index, s, e, params):
    p = params
    f32 = jnp.float32

    i_idx = edge_index[0]
    j_idx = edge_index[1]
    v16 = jnp.pad(v, ((0, 0), (0, 16 - v.shape[1])))

    i0, j0 = i_idx[:_EH0], j_idx[:_EH0]
    i1, j1 = i_idx[_EH0:], j_idx[_EH0:]
    si0, sj0, vi0, vj0 = _sc_gather(s, v16, i0, j0)
    si1, sj1, vi1, vj1 = _sc_gather(s, v16, i1, j1)

    rc = lambda a: a.reshape(-1, 1).astype(f32)
    w0T = p['e_W0'].T
    wlist = [
        w0T[:, 2:2 + _NH], w0T[:, 2 + _NH:2 + 2 * _NH], w0T[:, 2 + 2 * _NH:],
        w0T[:, 0:2],
        rc(p['e_b0']), rc(p['e_g0']), rc(p['e_be0']),
        p['e_W1'].T, rc(p['e_b1']),
        p['m_W0'].T, rc(p['m_b0']), rc(p['m_g0']), rc(p['m_be0']),
        p['m_W1'].T, rc(p['m_b1']),
        p['x_W0'].T, rc(p['x_b0']), rc(p['x_g0']), rc(p['x_be0']),
        p['x_W1'].T, rc(p['x_b1']), rc(p['x_g1']), rc(p['x_be1']),
        p['x_W2'].T, rc(p['x_b2']),
    ]
    eT = e.T
    eijT0, m0, u0 = _tc_dense(si0, sj0, vi0, vj0, eT[:, :_EH0], wlist)
    eijT1, m1, u1 = _tc_dense(si1, sj1, vi1, vj1, eT[:, _EH0:], wlist)

    z32 = jnp.zeros((_N, _NH), f32)
    z8 = jnp.zeros((_N, 8), f32)
    pm0 = _sc_scatter(m0, i0, z32, _NH)
    pu0 = _sc_scatter(u0, i0, z8, 8)
    pm1 = _sc_scatter(m1, i1, z32, _NH)
    pu1 = _sc_scatter(u1, i1, z8, 8)

    r1 = lambda a: a.reshape(1, -1).astype(f32)
    sw0 = p['s_W0']
    nlist = [
        sw0[:_NH], sw0[_NH:],
        r1(p['s_b0']), r1(p['s_g0']), r1(p['s_be0']),
        p['s_W1'], r1(p['s_b1']),
    ]
    v_t, s_t = _tc_node(v, s, pm0, pm1, pu0, pu1, nlist)
    eij = jnp.concatenate([eijT0, eijT1], axis=1).T
    return (v_t, s_t, eij)


# speedup vs baseline: 1.3858x; 1.3858x over previous
import functools

import jax
import jax.numpy as jnp
from jax import lax
from jax.experimental import pallas as pl
from jax.experimental.pallas import tpu as pltpu
from jax.experimental.pallas import tpu_sc as plsc

_N = 50000
_E = 800000
_NH = 32
_DE = 16

_EH0 = 403200
_EH1 = _E - _EH0

_NC = 2
_NS = 16
_NW = _NC * _NS

_SCH = 128

_NPS = _N // _NS

def _sc_gather(s_tab, v_tab, i_idx, j_idx):
    eh = i_idx.shape[0]
    epw = eh // _NW
    sfull = epw // _SCH
    stail = epw - sfull * _SCH

    def body(s_tab, v_tab, i_hbm, j_hbm, out_si, out_sj, out_vi, out_vj,
             *scr):
        idxi, idxj, si_v, sj_v, vi_v, vj_v = scr[0:6]
        sem1, sem2, sem3, sem4 = scr[-4:]
        c = lax.axis_index("c")
        sc = lax.axis_index("s")
        wid = sc * _NC + c
        e0 = wid * epw

        def step(base, idxi, idxj, si_v, sj_v, vi_v, vj_v, n):
            pltpu.sync_copy(i_hbm.at[pl.ds(base, n)], idxi)
            pltpu.sync_copy(j_hbm.at[pl.ds(base, n)], idxj)
            cp1 = pltpu.async_copy(s_tab.at[idxi], si_v, sem1)
            cp2 = pltpu.async_copy(s_tab.at[idxj], sj_v, sem2)
            cp3 = pltpu.async_copy(v_tab.at[idxi], vi_v, sem3)
            cp4 = pltpu.async_copy(v_tab.at[idxj], vj_v, sem4)
            cp1.wait()
            cp2.wait()
            cp3.wait()
            cp4.wait()
            pltpu.sync_copy(si_v, out_si.at[pl.ds(base, n)])
            pltpu.sync_copy(sj_v, out_sj.at[pl.ds(base, n)])
            pltpu.sync_copy(vi_v, out_vi.at[pl.ds(base, n)])
            pltpu.sync_copy(vj_v, out_vj.at[pl.ds(base, n)])

        def chunk(k, carry):
            step(e0 + _SCH * k, idxi, idxj, si_v, sj_v, vi_v, vj_v, _SCH)
            return carry

        lax.fori_loop(0, sfull, chunk, 0)
        if stail:
            step(e0 + _SCH * sfull, *scr[6:12], stail)

    scratch = [
        pltpu.VMEM((_SCH,), jnp.int32),
        pltpu.VMEM((_SCH,), jnp.int32),
        pltpu.VMEM((_SCH, _NH), jnp.float32),
        pltpu.VMEM((_SCH, _NH), jnp.float32),
        pltpu.VMEM((_SCH, _NH), jnp.float32),
        pltpu.VMEM((_SCH, _NH), jnp.float32),
    ]
    if stail:
        scratch += [
            pltpu.VMEM((stail,), jnp.int32),
            pltpu.VMEM((stail,), jnp.int32),
            pltpu.VMEM((stail, _NH), jnp.float32),
            pltpu.VMEM((stail, _NH), jnp.float32),
            pltpu.VMEM((stail, _NH), jnp.float32),
            pltpu.VMEM((stail, _NH), jnp.float32),
        ]
    scratch += [pltpu.SemaphoreType.DMA] * 4

    mesh = plsc.VectorSubcoreMesh(core_axis_name="c", subcore_axis_name="s")
    f = pl.kernel(
        body,
        out_type=(
            jax.ShapeDtypeStruct((eh, _NH), jnp.float32),
            jax.ShapeDtypeStruct((eh, _NH), jnp.float32),
            jax.ShapeDtypeStruct((eh, _NH), jnp.float32),
            jax.ShapeDtypeStruct((eh, _NH), jnp.float32),
        ),
        mesh=mesh,
        scratch_types=scratch,
        compiler_params=pltpu.CompilerParams(use_tc_tiling_on_sc=False),
    )
    return f(s_tab, v_tab, i_idx, j_idx)


def _sc_scatter(mu, i_idx, zeros, width):
    eh = i_idx.shape[0]
    epw = eh // _NW
    sfull = epw // _SCH
    stail = epw - sfull * _SCH

    def body(mu_hbm, i_hbm, z_hbm, out_hbm, *scr):
        idx_v, rows_v = scr[0:2]
        acc = scr[-1]
        c = lax.axis_index("c")
        sc = lax.axis_index("s")
        wid = sc * _NC + c
        e0 = wid * epw

        pltpu.sync_copy(z_hbm.at[pl.ds(sc * _NPS, _NPS)],
                        acc.at[pl.ds(sc * _NPS, _NPS)])
        plsc.subcore_barrier()

        def step(base, idx, rows, n):
            pltpu.sync_copy(i_hbm.at[pl.ds(base, n)], idx)
            pltpu.sync_copy(mu_hbm.at[pl.ds(base, n)], rows)
            pltpu.sync_copy(rows, acc.at[idx], add=True)

        def chunk(k, carry):
            step(e0 + _SCH * k, idx_v, rows_v, _SCH)
            return carry

        lax.fori_loop(0, sfull, chunk, 0)
        if stail:
            step(e0 + _SCH * sfull, scr[2], scr[3], stail)

        plsc.subcore_barrier()
        pltpu.sync_copy(acc.at[pl.ds(sc * _NPS, _NPS)],
                        out_hbm.at[c].at[pl.ds(sc * _NPS, _NPS)])

    scratch = [
        pltpu.VMEM((_SCH,), jnp.int32),
        pltpu.VMEM((_SCH, width), jnp.float32),
    ]
    if stail:
        scratch += [
            pltpu.VMEM((stail,), jnp.int32),
            pltpu.VMEM((stail, width), jnp.float32),
        ]
    scratch += [pltpu.VMEM_SHARED((_N, width), jnp.float32)]

    mesh = plsc.VectorSubcoreMesh(core_axis_name="c", subcore_axis_name="s")
    f = pl.kernel(
        body,
        out_type=jax.ShapeDtypeStruct((_NC, _N, width), jnp.float32),
        mesh=mesh,
        scratch_types=scratch,
        compiler_params=pltpu.CompilerParams(use_tc_tiling_on_sc=False),
    )
    return f(mu, i_idx, zeros)


def _lnorm(x, g, b):
    mu = jnp.mean(x, axis=-1, keepdims=True)
    var = jnp.mean(x * x, axis=-1, keepdims=True) - mu * mu
    return (x - mu) / jnp.sqrt(var + 1e-5) * g + b


def _lnorm4(x, g, b, mmean):
    dot = functools.partial(jnp.dot, preferred_element_type=jnp.float32)
    mu = dot(x, mmean)
    var = dot(x * x, mmean) - mu * mu
    return (x - mu) * lax.rsqrt(var + 1e-5) * g + b


def _dense_body(si4, sj4, vi4, vj4, e4,
                w0si4, w0sj4, w0e4, w0nd4, b0, g0, be0, w14, b1,
                mw04, mb0, mg0, mbe0, mw14, mb1,
                xw04, xb0, xg0, xbe0, xw14, xb1, xg1, xbe1, xw24, xb2,
                mmean, mn8, md8, bw4, r4, emask,
                eij_o, m_o, u_o):
    f32 = jnp.float32
    dot = functools.partial(jnp.dot, preferred_element_type=f32)

    vi = vi4[...]
    vj = vj4[...]
    vd = vi - vj
    nsq8 = dot(vd * vd, mn8[...]) + 1e-8 * emask[...]
    nd8 = emask[...] * jnp.sqrt(nsq8) + dot(vi * vj, md8[...])

    pre = (dot(si4[...], w0si4[...]) + dot(sj4[...], w0sj4[...])
           + dot(e4[...], w0e4[...]) + dot(nd8, w0nd4[...]) + b0[...])
    h = jnp.maximum(_lnorm4(pre, g0[...], be0[...], mmean[...]), 0.0)
    eij = dot(h, w14[...]) + b1[...]
    eij_o[...] = eij

    h = jnp.maximum(
        _lnorm4(dot(eij, mw04[...]) + mb0[...], mg0[...], mbe0[...], mmean[...]), 0.0)
    m = jax.nn.sigmoid(dot(h, mw14[...]) + mb1[...])
    m_o[...] = m

    h = jnp.maximum(
        _lnorm4(dot(m, xw04[...]) + xb0[...], xg0[...], xbe0[...], mmean[...]), 0.0)
    h = jnp.maximum(
        _lnorm4(dot(h, xw14[...]) + xb1[...], xg1[...], xbe1[...], mmean[...]), 0.0)
    w4 = dot(h, xw24[...]) + xb2[...]
    w128 = dot(w4, bw4[...])
    upd = jnp.clip(vd * w128, -100.0, 100.0)
    u_o[...] = dot(upd, r4[...])


_TILE_E = 3200
_T4 = _TILE_E // 4


def _tc_dense(si4, sj4, vi4, vj4, e4, wlist):
    ne4 = si4.shape[0]
    grid = ne4 // _T4
    full = lambda a: pl.BlockSpec(a.shape, lambda i: (0, 0))
    in_specs = [
        pl.BlockSpec((_T4, 128), lambda i: (i, 0)),
        pl.BlockSpec((_T4, 128), lambda i: (i, 0)),
        pl.BlockSpec((_T4, 128), lambda i: (i, 0)),
        pl.BlockSpec((_T4, 128), lambda i: (i, 0)),
        pl.BlockSpec((_T4, 64), lambda i: (i, 0)),
    ] + [full(w) for w in wlist]
    out_specs = [
        pl.BlockSpec((_T4, 128), lambda i: (i, 0)),
        pl.BlockSpec((_T4, 128), lambda i: (i, 0)),
        pl.BlockSpec((_T4, 32), lambda i: (i, 0)),
    ]
    return pl.pallas_call(
        _dense_body,
        grid=(grid,),
        in_specs=in_specs,
        out_specs=out_specs,
        out_shape=[
            jax.ShapeDtypeStruct((ne4, 128), jnp.float32),
            jax.ShapeDtypeStruct((ne4, 128), jnp.float32),
            jax.ShapeDtypeStruct((ne4, 32), jnp.float32),
        ],
        compiler_params=pltpu.CompilerParams(
            dimension_semantics=("arbitrary",),
        ),
    )(si4, sj4, vi4, vj4, e4, *wlist)


def _node_body(v_r, s_r, pm0_r, pm1_r, pu0_r, pu1_r,
               sw0a, sw0b, sb0, sg0, sbe0, sw1, sb1,
               vt_o, st_o):
    f32 = jnp.float32
    s_agg = pm0_r[0] + pm0_r[1] + pm1_r[0] + pm1_r[1]
    x_agg = (pu0_r[0] + pu0_r[1] + pu1_r[0] + pu1_r[1])[:, 0:3]
    s = s_r[...]
    dotf = functools.partial(jnp.dot, preferred_element_type=f32)
    pre = dotf(s, sw0a[...]) + dotf(s_agg, sw0b[...]) + sb0[...]
    h = jnp.maximum(_lnorm(pre, sg0[...], sbe0[...]), 0.0)
    st_o[...] = s + dotf(h, sw1[...]) + sb1[...]
    vt_o[...] = v_r[...] + x_agg


_TILE_N = 2000


def _tc_node(v, s, pm0, pm1, pu0, pu1, wlist):
    grid = _N // _TILE_N
    full = lambda a: pl.BlockSpec(a.shape, lambda i: (0, 0))
    in_specs = [
        pl.BlockSpec((_TILE_N, 3), lambda i: (i, 0)),
        pl.BlockSpec((_TILE_N, _NH), lambda i: (i, 0)),
        pl.BlockSpec((_NC, _TILE_N, _NH), lambda i: (0, i, 0)),
        pl.BlockSpec((_NC, _TILE_N, _NH), lambda i: (0, i, 0)),
        pl.BlockSpec((_NC, _TILE_N, 8), lambda i: (0, i, 0)),
        pl.BlockSpec((_NC, _TILE_N, 8), lambda i: (0, i, 0)),
    ] + [full(w) for w in wlist]
    out_specs = [
        pl.BlockSpec((_TILE_N, 3), lambda i: (i, 0)),
        pl.BlockSpec((_TILE_N, _NH), lambda i: (i, 0)),
    ]
    return pl.pallas_call(
        _node_body,
        grid=(grid,),
        in_specs=in_specs,
        out_specs=out_specs,
        out_shape=[
            jax.ShapeDtypeStruct((_N, 3), jnp.float32),
            jax.ShapeDtypeStruct((_N, _NH), jnp.float32),
        ],
        compiler_params=pltpu.CompilerParams(
            dimension_semantics=("arbitrary",),
        ),
    )(v, s, pm0, pm1, pu0, pu1, *wlist)


def kernel(v, edge_index, s, e, params):
    p = params
    f32 = jnp.float32

    i_idx = edge_index[0]
    j_idx = edge_index[1]
    v32 = jnp.pad(v, ((0, 0), (0, _NH - v.shape[1])))

    i0, j0 = i_idx[:_EH0], j_idx[:_EH0]
    i1, j1 = i_idx[_EH0:], j_idx[_EH0:]
    si0, sj0, vi0, vj0 = _sc_gather(s, v32, i0, j0)
    si1, sj1, vi1, vj1 = _sc_gather(s, v32, i1, j1)

    I4 = jnp.eye(4, dtype=f32)
    bd = lambda w: jnp.kron(I4, w.astype(f32))
    t4 = lambda a: jnp.tile(a.astype(f32).reshape(1, -1), (1, 4))
    W0 = p['e_W0']
    mmean = jnp.kron(I4, jnp.full((_NH, _NH), 1.0 / _NH, f32))
    cn = jnp.zeros((_NH, 2), f32).at[0:3, 0].set(1.0)
    cd = jnp.zeros((_NH, 2), f32).at[0:3, 1].set(1.0)
    mn8 = jnp.kron(I4, cn)
    md8 = jnp.kron(I4, cd)
    emask = jnp.tile(jnp.array([[1.0, 0.0]], f32), (1, 4))
    bw4 = jnp.kron(I4, jnp.ones((1, _NH), f32))
    rr = jnp.zeros((_NH, 8), f32).at[0, 0].set(1.0).at[1, 1].set(1.0).at[2, 2].set(1.0)
    r4 = jnp.kron(I4, rr)
    wlist = [
        bd(W0[2:2 + _NH]), bd(W0[2 + _NH:2 + 2 * _NH]), bd(W0[2 + 2 * _NH:]),
        bd(W0[0:2]),
        t4(p['e_b0']), t4(p['e_g0']), t4(p['e_be0']),
        bd(p['e_W1']), t4(p['e_b1']),
        bd(p['m_W0']), t4(p['m_b0']), t4(p['m_g0']), t4(p['m_be0']),
        bd(p['m_W1']), t4(p['m_b1']),
        bd(p['x_W0']), t4(p['x_b0']), t4(p['x_g0']), t4(p['x_be0']),
        bd(p['x_W1']), t4(p['x_b1']), t4(p['x_g1']), t4(p['x_be1']),
        bd(p['x_W2']), t4(p['x_b2']),
        mmean, mn8, md8, bw4, r4, emask,
    ]
    p4 = lambda a: a.reshape(-1, 4 * a.shape[1])
    eij0, m0, u0 = _tc_dense(p4(si0), p4(sj0), p4(vi0), p4(vj0),
                             p4(e[:_EH0]), wlist)
    eij1, m1, u1 = _tc_dense(p4(si1), p4(sj1), p4(vi1), p4(vj1),
                             p4(e[_EH0:]), wlist)

    z32 = jnp.zeros((_N, _NH), f32)
    z8 = jnp.zeros((_N, 8), f32)
    pm0 = _sc_scatter(m0.reshape(-1, _NH), i0, z32, _NH)
    pu0 = _sc_scatter(u0.reshape(-1, 8), i0, z8, 8)
    pm1 = _sc_scatter(m1.reshape(-1, _NH), i1, z32, _NH)
    pu1 = _sc_scatter(u1.reshape(-1, 8), i1, z8, 8)

    r1 = lambda a: a.reshape(1, -1).astype(f32)
    sw0 = p['s_W0']
    nlist = [
        sw0[:_NH], sw0[_NH:],
        r1(p['s_b0']), r1(p['s_g0']), r1(p['s_be0']),
        p['s_W1'], r1(p['s_b1']),
    ]
    v_t, s_t = _tc_node(v, s, pm0, pm1, pu0, pu1, nlist)
    eij = jnp.concatenate([eij0.reshape(-1, _NH), eij1.reshape(-1, _NH)],
                          axis=0)
    return (v_t, s_t, eij)
